# baseline (speedup 1.0000x reference)
import jax
import jax.numpy as jnp
from jax.experimental import pallas as pl
from jax.experimental.pallas import tpu as pltpu

_S = 8


def _pool_body(x_ref, o_ref):
    j = pl.program_id(1)

    @pl.when(j == 0)
    def _():
        o_ref[...] = jnp.zeros_like(o_ref)

    x = x_ref[...]
    o_ref[...] += jnp.sum(x.reshape(-1, 8, x.shape[-1]), axis=0)


def kernel(features, h, w):
    n, c = features.shape
    g = 128 // c if (c < 128 and 128 % c == 0) else 1
    lanes = c * g
    rows = n // g if n % g == 0 else 0
    if rows % (_S * 8) != 0:
        rows = 0
    if rows == 0:
        target = -(-n // (g * _S * 8)) * (g * _S * 8)
        features = jnp.pad(features, ((0, target - n), (0, 0)))
        n = target
        rows = n // g
    xr = features.reshape(rows, lanes)

    half = rows // 2
    k = 1
    for cand in range(40, 0, -1):
        if (half // 8) % cand == 0:
            k = cand
            break
    bn = half // k

    partials = pl.pallas_call(
        _pool_body,
        grid=(2, k),
        in_specs=[pl.BlockSpec((bn, lanes), lambda i, j: (i * k + j, 0))],
        out_specs=pl.BlockSpec((8, lanes), lambda i, j: (i, 0)),
        out_shape=jax.ShapeDtypeStruct((16, lanes), jnp.float32),
        compiler_params=pltpu.CompilerParams(
            dimension_semantics=("parallel", "arbitrary"),
        ),
    )(xr)
    total = jnp.sum(partials, axis=0).reshape(g, c).sum(axis=0)
    return total / (h * w)


# speedup vs baseline: 1.8679x; 1.8679x over previous
import functools

import jax
import jax.numpy as jnp
from jax.experimental import pallas as pl
from jax.experimental.pallas import tpu as pltpu

_S = 8
_D = 4


def _pool_body(x_hbm, o_ref, buf, sem, *, k, bn):
    i = pl.program_id(0)
    j = pl.program_id(1)
    sh = _S // 2

    def start(slot, step):
        pltpu.make_async_copy(
            x_hbm.at[pl.ds(i * sh, sh), pl.ds(step * bn, bn), :],
            buf.at[slot],
            sem.at[slot],
        ).start()

    @pl.when(j == 0)
    def _():
        for d in range(min(_D, k)):
            start(d, d)

    @pl.when(j == 0)
    def _():
        o_ref[...] = jnp.zeros_like(o_ref)

    slot = j % _D
    pltpu.make_async_copy(
        x_hbm.at[pl.ds(i * sh, sh), pl.ds(j * bn, bn), :],
        buf.at[slot],
        sem.at[slot],
    ).wait()

    x = buf[slot]
    o_ref[...] += jnp.sum(x, axis=1)[None]

    if k > _D:

        @pl.when(j + _D < k)
        def _():
            start(slot, j + _D)


def kernel(features, h, w):
    n, c = features.shape
    if n % (_S * 8) != 0:
        target = -(-n // (_S * 8)) * (_S * 8)
        features = jnp.pad(features, ((0, target - n), (0, 0)))
        n = target
    stripe = n // _S
    k = 1
    for cand in range(64, 0, -1):
        if (stripe // 8) % cand == 0:
            k = cand
            break
    bn = stripe // k
    xr = features.reshape(_S, stripe, c)

    body = functools.partial(_pool_body, k=k, bn=bn)
    partials = pl.pallas_call(
        body,
        grid=(2, k),
        in_specs=[pl.BlockSpec(memory_space=pl.ANY)],
        out_specs=pl.BlockSpec((1, _S // 2, c), lambda i, j: (i, 0, 0)),
        out_shape=jax.ShapeDtypeStruct((2, _S // 2, c), jnp.float32),
        scratch_shapes=[
            pltpu.VMEM((_D, _S // 2, bn, c), jnp.float32),
            pltpu.SemaphoreType.DMA((_D,)),
        ],
        compiler_params=pltpu.CompilerParams(
            dimension_semantics=("parallel", "arbitrary"),
        ),
    )(xr)
    return jnp.sum(partials, axis=(0, 1)) / (h * w)


# speedup vs baseline: 1.8980x; 1.0161x over previous
import functools

import jax
import jax.numpy as jnp
from jax.experimental import pallas as pl
from jax.experimental.pallas import tpu as pltpu

_S = 16
_D = 6


def _pool_body(x_hbm, o_ref, buf, sem, *, k, bn):
    i = pl.program_id(0)
    j = pl.program_id(1)
    sh = _S // 2

    def start(slot, step, prio):
        pltpu.make_async_copy(
            x_hbm.at[pl.ds(i * sh, sh), pl.ds(step * bn, bn), :],
            buf.at[slot],
            sem.at[slot],
        ).start(priority=prio)

    @pl.when(j == 0)
    def _():
        for d in range(min(_D, k)):
            start(d, d, d % 2)

    @pl.when(j == 0)
    def _():
        o_ref[...] = jnp.zeros_like(o_ref)

    slot = j % _D
    pltpu.make_async_copy(
        x_hbm.at[pl.ds(i * sh, sh), pl.ds(j * bn, bn), :],
        buf.at[slot],
        sem.at[slot],
    ).wait()

    x = buf[slot]
    o_ref[...] += jnp.sum(x, axis=1)[None]

    if k > _D:

        nxt = j + _D
        @pl.when(jnp.logical_and(nxt < k, nxt % 2 == 0))
        def _():
            start(slot, nxt, 0)

        @pl.when(jnp.logical_and(nxt < k, nxt % 2 == 1))
        def _():
            start(slot, nxt, 1)


def kernel(features, h, w):
    n, c = features.shape
    if n % (_S * 8) != 0:
        target = -(-n // (_S * 8)) * (_S * 8)
        features = jnp.pad(features, ((0, target - n), (0, 0)))
        n = target
    stripe = n // _S
    k = stripe // 8
    for cand in range(1, stripe // 8 + 1):
        if (stripe // 8) % cand == 0 and stripe // cand <= 1600:
            k = cand
            break
    bn = stripe // k
    xr = features.reshape(_S, stripe, c)

    body = functools.partial(_pool_body, k=k, bn=bn)
    partials = pl.pallas_call(
        body,
        grid=(2, k),
        in_specs=[pl.BlockSpec(memory_space=pl.ANY)],
        out_specs=pl.BlockSpec((1, _S // 2, c), lambda i, j: (i, 0, 0)),
        out_shape=jax.ShapeDtypeStruct((2, _S // 2, c), jnp.float32),
        scratch_shapes=[
            pltpu.VMEM((_D, _S // 2, bn, c), jnp.float32),
            pltpu.SemaphoreType.DMA((_D,)),
        ],
        compiler_params=pltpu.CompilerParams(
            dimension_semantics=("parallel", "arbitrary"),
        ),
    )(xr)
    return jnp.sum(partials, axis=(0, 1)) / (h * w)
